# Initial kernel scaffold; baseline (speedup 1.0000x reference)
import jax
import jax.numpy as jnp
from jax.experimental import pallas as pl


def kernel(hidden_states, gate_w, w_gate_up, w_down, shared_gate_up, shared_down):
    raise NotImplementedError("write your pallas kernel here")


# speedup vs baseline: 1.8833x; 1.8833x over previous
import functools

import jax
import jax.numpy as jnp
from jax.experimental import pallas as pl
from jax.experimental.pallas import tpu as pltpu

E = 8
TOP_K = 2
D_MODEL = 1024
MOE_FF = 512
SHARED_FF = 1024
T = 2048

TOKEN_TILE = 256


def _silu(x):
    return x * jax.nn.sigmoid(x)


def _fused_moe_body(x_ref, gate_wt_ref, sgu_ref, sdown_ref, wgu_ref, wdown_ref,
                    out_ref):
    x = x_ref[...]
    xb = x.astype(jnp.bfloat16)

    h = jnp.dot(xb, sgu_ref[...], preferred_element_type=jnp.float32)
    g = h[:, :SHARED_FF]
    u = h[:, SHARED_FF:]
    act = (_silu(g) * u).astype(jnp.bfloat16)
    acc = jnp.dot(act, sdown_ref[...], preferred_element_type=jnp.float32)

    logits = jax.lax.dot_general(
        x, gate_wt_ref[...], (((1,), (0,)), ((), ())),
        preferred_element_type=jnp.float32)
    m = jnp.max(logits, axis=-1, keepdims=True)
    ex = jnp.exp(logits - m)
    probs = ex / jnp.sum(ex, axis=-1, keepdims=True)

    iota = jax.lax.broadcasted_iota(jnp.int32, probs.shape, 1)
    p1 = jnp.max(probs, axis=-1, keepdims=True)
    id1 = jnp.min(jnp.where(probs == p1, iota, E), axis=-1, keepdims=True)
    oh1 = iota == id1
    probs_m = jnp.where(oh1, -jnp.inf, probs)
    p2 = jnp.max(probs_m, axis=-1, keepdims=True)
    id2 = jnp.min(jnp.where(probs_m == p2, iota, E), axis=-1, keepdims=True)
    oh2 = iota == id2
    denom = p1 + p2
    combine = (oh1 * (p1 / denom) + oh2 * (p2 / denom)).astype(jnp.float32)

    for e in range(E):
        he = jnp.dot(xb, wgu_ref[e], preferred_element_type=jnp.float32)
        ge = he[:, :MOE_FF]
        ue = he[:, MOE_FF:]
        ae = (_silu(ge) * ue).astype(jnp.bfloat16)
        ye = jnp.dot(ae, wdown_ref[e], preferred_element_type=jnp.float32)
        acc += combine[:, e:e + 1] * ye

    out_ref[...] = acc


@jax.jit
def kernel(hidden_states, gate_w, w_gate_up, w_down, shared_gate_up,
           shared_down):
    num_tokens, d = hidden_states.shape
    grid = num_tokens // TOKEN_TILE

    gate_wt = gate_w.T
    sgu = shared_gate_up.astype(jnp.bfloat16)
    sdown = shared_down.astype(jnp.bfloat16)
    wgu = w_gate_up.astype(jnp.bfloat16)
    wdown = w_down.astype(jnp.bfloat16)

    full = lambda *shape: pl.BlockSpec(shape, lambda i: (0,) * len(shape))
    out = pl.pallas_call(
        _fused_moe_body,
        grid=(grid,),
        in_specs=[
            pl.BlockSpec((TOKEN_TILE, d), lambda i: (i, 0)),
            full(d, E),
            full(d, 2 * SHARED_FF),
            full(SHARED_FF, d),
            full(E, d, 2 * MOE_FF),
            full(E, MOE_FF, d),
        ],
        out_specs=pl.BlockSpec((TOKEN_TILE, d), lambda i: (i, 0)),
        out_shape=jax.ShapeDtypeStruct((num_tokens, d), jnp.float32),
    )(hidden_states, gate_wt, sgu, sdown, wgu, wdown)
    return out
